# baseline (speedup 1.0000x reference)
import functools

import jax
import jax.numpy as jnp
from jax import lax
from jax.experimental import pallas as pl
from jax.experimental.pallas import tpu as pltpu
from jax.experimental.pallas import tpu_sc as plsc

VOCAB = 8192
DIM = 8192
HALF = DIM // 2
NLOOKUP = 8192
BPG = 512
CHUNK = 8
NGSTEP = 2 * (BPG // CHUNK)
NPUSH = 128

_mesh = plsc.VectorSubcoreMesh(core_axis_name="c", subcore_axis_name="s")


@functools.partial(
    pl.kernel,
    mesh=_mesh,
    out_type=jax.ShapeDtypeStruct((NLOOKUP, DIM), jnp.float32),
    scratch_types=[
        pltpu.VMEM((BPG,), jnp.int32),
        pltpu.VMEM((CHUNK, HALF), jnp.float32),
        pltpu.VMEM((CHUNK, HALF), jnp.float32),
        pltpu.VMEM_SHARED((7, CHUNK, HALF), jnp.float32),
        pltpu.SemaphoreType.DMA,
        pltpu.SemaphoreType.DMA,
    ],
)
def _probe(idx_hbm, table_hbm, out_hbm, idx_v, buf0, buf1, shared, s0, s1):
    core = lax.axis_index("c")
    sub = lax.axis_index("s")
    bufs = (buf0, buf1)
    sems = (s0, s1)

    @pl.when(sub < 8)
    def _gather_role():
        base = (core * 8 + sub) * BPG
        pltpu.sync_copy(idx_hbm.at[pl.ds(base, BPG)], idx_v)

        def start_gather(s, b):
            c = s // 2
            h = s % 2
            pltpu.async_copy(
                table_hbm.at[idx_v.at[pl.ds(c * CHUNK, CHUNK)],
                             pl.ds(h * HALF, HALF)],
                bufs[b], sems[b],
            )

        def wait(sem):
            pltpu.make_async_copy(
                out_hbm.at[pl.ds(0, CHUNK), pl.ds(0, HALF)], bufs[0], sem
            ).wait()

        start_gather(0, 0)
        start_gather(1, 1)

        def body(k, carry):
            for b in range(2):
                s = 2 * k + b
                wait(sems[b])
                start_gather(s + 2, b)
            return carry

        lax.fori_loop(0, NGSTEP // 2 - 1, body, 0)
        wait(sems[0])
        wait(sems[1])
        pltpu.sync_copy(bufs[0], out_hbm.at[pl.ds(base, CHUNK),
                                            pl.ds(0, HALF)])

    @pl.when(sub >= 8)
    def _push_role():
        slot = lax.rem(sub, 7)

        def start_push(b):
            pltpu.async_copy(bufs[b], shared.at[slot], sems[b])

        def wait_p(sem):
            pltpu.make_async_copy(bufs[0], shared.at[0], sem).wait()

        start_push(0)
        start_push(1)

        def body(k, carry):
            for b in range(2):
                wait_p(sems[b])
                start_push(b)
            return carry

        lax.fori_loop(0, NPUSH // 2 - 1, body, 0)
        wait_p(sems[0])
        wait_p(sems[1])


def kernel(idx, table):
    flat_idx = idx.reshape(-1).astype(jnp.int32)
    out = _probe(flat_idx, table)
    return out.reshape(idx.shape[0], idx.shape[1], DIM)


# speedup vs baseline: 1.1979x; 1.1979x over previous
import functools

import jax
import jax.numpy as jnp
from jax import lax
from jax.experimental import pallas as pl
from jax.experimental.pallas import tpu as pltpu
from jax.experimental.pallas import tpu_sc as plsc

VOCAB = 8192
DIM = 8192
HALF = DIM // 2
NLOOKUP = 8192
NWORKER = 32
BPW = NLOOKUP // NWORKER
CHUNK = 8
NSTEP = 2 * (BPW // CHUNK)

_mesh = plsc.VectorSubcoreMesh(core_axis_name="c", subcore_axis_name="s")


@functools.partial(
    pl.kernel,
    mesh=_mesh,
    out_type=jax.ShapeDtypeStruct((NLOOKUP, DIM), jnp.float32),
    scratch_types=[
        pltpu.VMEM((BPW,), jnp.int32),
        pltpu.VMEM((CHUNK, HALF), jnp.float32),
        pltpu.VMEM((CHUNK, HALF), jnp.float32),
        pltpu.SemaphoreType.DMA,
        pltpu.SemaphoreType.DMA,
        pltpu.SemaphoreType.DMA,
        pltpu.SemaphoreType.DMA,
    ],
)
def _gather(idx_hbm, table_hbm, out_hbm, idx_v, buf0, buf1,
            gsem0, gsem1, osem0, osem1):
    wid = lax.axis_index("s") * 2 + lax.axis_index("c")
    base = wid * BPW
    pltpu.sync_copy(idx_hbm.at[pl.ds(base, BPW)], idx_v)

    bufs = (buf0, buf1)
    gsems = (gsem0, gsem1)
    osems = (osem0, osem1)

    def start_gather(s, b):
        c = s // 2
        h = s % 2
        pltpu.async_copy(
            table_hbm.at[idx_v.at[pl.ds(c * CHUNK, CHUNK)],
                         pl.ds(h * HALF, HALF)],
            bufs[b], gsems[b],
        )

    def start_out(s, b):
        c = s // 2
        h = s % 2
        pltpu.async_copy(
            bufs[b],
            out_hbm.at[pl.ds(base + c * CHUNK, CHUNK), pl.ds(h * HALF, HALF)],
            osems[b],
        )

    def wait(sem):
        pltpu.make_async_copy(
            out_hbm.at[pl.ds(0, CHUNK), pl.ds(0, HALF)], bufs[0], sem
        ).wait()

    start_gather(0, 0)
    start_gather(1, 1)

    def body(k, carry):
        for b in range(2):
            s = 2 * k + b
            wait(gsems[b])
            start_out(s, b)
            wait(osems[b])
            start_gather(s + 2, b)
        return carry

    lax.fori_loop(0, NSTEP // 2 - 1, body, 0)

    for b in range(2):
        s = NSTEP - 2 + b
        wait(gsems[b])
        start_out(s, b)
        wait(osems[b])


def kernel(idx, table):
    flat_idx = idx.reshape(-1).astype(jnp.int32)
    out = _gather(flat_idx, table)
    return out.reshape(idx.shape[0], idx.shape[1], DIM)
